# baseline (speedup 1.0000x reference)
import jax
import jax.numpy as jnp
from jax.experimental import pallas as pl
from jax.experimental.pallas import tpu as pltpu

D_MODEL = 1024
D_SPACE = 64
S = 2048
N_TOTAL = 16384
N_OUT = 20480
TN = 1024
TS = 256
NUM_J = N_OUT // TN
NUM_I = S // TS

_HTAB = (0, 0, 1, 1, 2, 2, 3, 3, 3, 3, 4, 4, 5, 5, 6, 6, 7, 7, 7, 7)
_NROW = (0, 1024, 0, 1024, 2048, 3072, 8192, 9216, 10240, 11264,
         4096, 5120, 4096, 5120, 6144, 7168, 12288, 13312, 14336, 15360)


def _ln_into(scr, k, t, g_ref, b_ref):
    g = g_ref[:, k * D_SPACE:(k + 1) * D_SPACE]
    b = b_ref[:, k * D_SPACE:(k + 1) * D_SPACE]
    m = jnp.mean(t, axis=-1, keepdims=True)
    v = jnp.mean((t - m) ** 2, axis=-1, keepdims=True)
    scr[k] = ((t - m) * jax.lax.rsqrt(v + 1e-5) * g + b).astype(jnp.bfloat16)


def _body(tab_ref, x_ref, ca_ref, ck_ref, ne_ref, Wx_ref, bx_ref, Wr_ref,
          br_ref, Wkn_ref, bkn_ref, g_ref, beta_ref, out_ref, h_scr):
    j = pl.program_id(1)

    @pl.when(j == 0)
    def _prologue():
        px = jnp.dot(x_ref[...], Wx_ref[...],
                     preferred_element_type=jnp.float32) + bx_ref[...]
        pr = jnp.dot(ca_ref[...], Wr_ref[...],
                     preferred_element_type=jnp.float32) + br_ref[...]
        pk = jnp.dot(ck_ref[...], Wkn_ref[...],
                     preferred_element_type=jnp.float32) + bkn_ref[...]
        for k in range(4):
            _ln_into(h_scr, k, px[:, k * D_SPACE:(k + 1) * D_SPACE],
                     g_ref, beta_ref)
        for k in range(3):
            _ln_into(h_scr, 4 + k, pr[:, k * D_SPACE:(k + 1) * D_SPACE],
                     g_ref, beta_ref)
        _ln_into(h_scr, 7, pk, g_ref, beta_ref)

    e = ne_ref[pl.ds(tab_ref[0, j], TN), :]
    inv = 1.0 / jnp.maximum(
        jnp.sqrt(jnp.sum(e * e, axis=-1, keepdims=True)), 1e-12)
    en = (e * inv).astype(jnp.bfloat16)
    h = h_scr[tab_ref[1, j]]
    out_ref[...] = jax.lax.dot_general(
        h, en, (((1,), (1,)), ((), ())), preferred_element_type=jnp.float32)


def kernel(x, ctx_attn, ctx_know, neuron_emb, W_feat, b_feat, W_know, b_know,
           W_rQ, b_rQ, W_rK, b_rK, W_rV, b_rV, W_rKn, b_rKn,
           g_fqkQ, beta_fqkQ, g_fqkK, beta_fqkK, g_fv, beta_fv,
           g_fkn, beta_fkn, g_rQ, beta_rQ, g_rK, beta_rK,
           g_rV, beta_rV, g_rKn, beta_rKn):
    B = x.shape[0]
    x2 = x.reshape(B * S, D_MODEL)
    ca = ctx_attn.reshape(B * S, -1)
    ck = ctx_know.reshape(B * S, -1)

    Wx = jnp.concatenate([W_feat, W_know], axis=1)
    bx = jnp.concatenate([b_feat, b_know])[None, :]
    Wr = jnp.concatenate([W_rQ, W_rK, W_rV], axis=1)
    br = jnp.concatenate([b_rQ, b_rK, b_rV])[None, :]
    bkn = b_rKn[None, :]
    g = jnp.concatenate([g_fqkQ, g_fqkK, g_fv, g_fkn,
                         g_rQ, g_rK, g_rV, g_rKn])[None, :]
    beta = jnp.concatenate([beta_fqkQ, beta_fqkK, beta_fv, beta_fkn,
                            beta_rQ, beta_rK, beta_rV, beta_rKn])[None, :]

    tab = jnp.asarray([_NROW, _HTAB], dtype=jnp.int32)
    full = lambda a: pl.BlockSpec(a.shape, lambda i, j, t: (0,) * a.ndim)
    tile = lambda a: pl.BlockSpec((TS, a.shape[1]), lambda i, j, t: (i, 0))

    grid_spec = pltpu.PrefetchScalarGridSpec(
        num_scalar_prefetch=1,
        grid=(NUM_I, NUM_J),
        in_specs=[
            tile(x2), tile(ca), tile(ck),
            full(neuron_emb),
            full(Wx), full(bx), full(Wr), full(br),
            full(W_rKn), full(bkn), full(g), full(beta),
        ],
        out_specs=pl.BlockSpec((TS, TN), lambda i, j, t: (i, j)),
        scratch_shapes=[pltpu.VMEM((8, TS, D_SPACE), jnp.bfloat16)],
    )

    out = pl.pallas_call(
        _body,
        grid_spec=grid_spec,
        out_shape=jax.ShapeDtypeStruct((B * S, N_OUT), jnp.float32),
    )(tab, x2, ca, ck, neuron_emb, Wx, bx, Wr, br, W_rKn, bkn, g, beta)

    return out.reshape(B, S, N_OUT)


# speedup vs baseline: 1.7273x; 1.7273x over previous
import jax
import jax.numpy as jnp
from jax.experimental import pallas as pl
from jax.experimental.pallas import tpu as pltpu

D_MODEL = 1024
D_SPACE = 64
S = 2048
N_TOTAL = 16384
N_OUT = 20480
TN = 1024
NUM_J = N_OUT // TN

_HTAB = (0, 0, 1, 1, 2, 2, 3, 3, 3, 3, 4, 4, 5, 5, 6, 6, 7, 7, 7, 7)
_NROW = (0, 1024, 0, 1024, 2048, 3072, 8192, 9216, 10240, 11264,
         4096, 5120, 4096, 5120, 6144, 7168, 12288, 13312, 14336, 15360)


def _ln_into(scr, k, t, g_ref, b_ref):
    g = g_ref[:, k * D_SPACE:(k + 1) * D_SPACE]
    b = b_ref[:, k * D_SPACE:(k + 1) * D_SPACE]
    m = jnp.mean(t, axis=-1, keepdims=True)
    v = jnp.mean((t - m) ** 2, axis=-1, keepdims=True)
    scr[k] = ((t - m) * jax.lax.rsqrt(v + 1e-5) * g + b).astype(jnp.bfloat16)


def _body(tab_ref, x_ref, ca_ref, ck_ref, ne_ref, Wx_ref, bx_ref, Wr_ref,
          br_ref, Wkn_ref, bkn_ref, g_ref, beta_ref, out_ref, h_scr, ne_scr):
    j = pl.program_id(0)

    @pl.when(j == 0)
    def _prologue():
        px = jnp.dot(x_ref[...], Wx_ref[...],
                     preferred_element_type=jnp.float32) + bx_ref[...]
        pr = jnp.dot(ca_ref[...], Wr_ref[...],
                     preferred_element_type=jnp.float32) + br_ref[...]
        pk = jnp.dot(ck_ref[...], Wkn_ref[...],
                     preferred_element_type=jnp.float32) + bkn_ref[...]
        for k in range(4):
            _ln_into(h_scr, k, px[:, k * D_SPACE:(k + 1) * D_SPACE],
                     g_ref, beta_ref)
        for k in range(3):
            _ln_into(h_scr, 4 + k, pr[:, k * D_SPACE:(k + 1) * D_SPACE],
                     g_ref, beta_ref)
        _ln_into(h_scr, 7, pk, g_ref, beta_ref)
        e = ne_ref[...]
        inv = 1.0 / jnp.maximum(
            jnp.sqrt(jnp.sum(e * e, axis=-1, keepdims=True)), 1e-12)
        ne_scr[...] = (e * inv).astype(jnp.bfloat16)

    @pl.when(j > 0)
    def _main():
        jj = j - 1
        row = pl.multiple_of(tab_ref[0, jj], TN)
        en = ne_scr[pl.ds(row, TN), :]
        h = h_scr[tab_ref[1, jj]]
        out_ref[...] = jax.lax.dot_general(
            h, en, (((1,), (1,)), ((), ())),
            preferred_element_type=jnp.float32)


def kernel(x, ctx_attn, ctx_know, neuron_emb, W_feat, b_feat, W_know, b_know,
           W_rQ, b_rQ, W_rK, b_rK, W_rV, b_rV, W_rKn, b_rKn,
           g_fqkQ, beta_fqkQ, g_fqkK, beta_fqkK, g_fv, beta_fv,
           g_fkn, beta_fkn, g_rQ, beta_rQ, g_rK, beta_rK,
           g_rV, beta_rV, g_rKn, beta_rKn):
    B = x.shape[0]
    x2 = x.reshape(B * S, D_MODEL)
    ca = ctx_attn.reshape(B * S, -1)
    ck = ctx_know.reshape(B * S, -1)

    Wx = jnp.concatenate([W_feat, W_know], axis=1)
    bx = jnp.concatenate([b_feat, b_know])[None, :]
    Wr = jnp.concatenate([W_rQ, W_rK, W_rV], axis=1)
    br = jnp.concatenate([b_rQ, b_rK, b_rV])[None, :]
    bkn = b_rKn[None, :]
    g = jnp.concatenate([g_fqkQ, g_fqkK, g_fv, g_fkn,
                         g_rQ, g_rK, g_rV, g_rKn])[None, :]
    beta = jnp.concatenate([beta_fqkQ, beta_fqkK, beta_fv, beta_fkn,
                            beta_rQ, beta_rK, beta_rV, beta_rKn])[None, :]

    tab = jnp.asarray([_NROW, _HTAB], dtype=jnp.int32)
    full = lambda a: pl.BlockSpec(a.shape, lambda j, t: (0,) * a.ndim)

    grid_spec = pltpu.PrefetchScalarGridSpec(
        num_scalar_prefetch=1,
        grid=(NUM_J + 1,),
        in_specs=[
            full(x2), full(ca), full(ck), full(neuron_emb),
            full(Wx), full(bx), full(Wr), full(br),
            full(W_rKn), full(bkn), full(g), full(beta),
        ],
        out_specs=pl.BlockSpec((B * S, TN),
                               lambda j, t: (0, jnp.maximum(j - 1, 0))),
        scratch_shapes=[pltpu.VMEM((8, B * S, D_SPACE), jnp.bfloat16),
                        pltpu.VMEM((N_TOTAL, D_SPACE), jnp.bfloat16)],
    )

    out = pl.pallas_call(
        _body,
        grid_spec=grid_spec,
        out_shape=jax.ShapeDtypeStruct((B * S, N_OUT), jnp.float32),
    )(tab, x2, ca, ck, neuron_emb, Wx, bx, Wr, br, W_rKn, bkn, g, beta)

    return out.reshape(B, S, N_OUT)
